# Initial kernel scaffold; baseline (speedup 1.0000x reference)
import jax
import jax.numpy as jnp
from jax.experimental import pallas as pl


def kernel(x, idx, shift):
    raise NotImplementedError("write your pallas kernel here")


# speedup vs baseline: 1.2533x; 1.2533x over previous
import functools

import jax
import jax.numpy as jnp
from jax import lax
from jax.experimental import pallas as pl
from jax.experimental.pallas import tpu as pltpu
from jax.experimental.pallas import tpu_sc as plsc

_NC = 2
_NS = 16
_NW = _NC * _NS
_L = 16


@functools.lru_cache(maxsize=None)
def _make_gather(N, D, B):
    b_per_w = B // _NW
    C = 128
    n_chunks = b_per_w // C
    mesh = plsc.VectorSubcoreMesh(core_axis_name="c", subcore_axis_name="s")

    @functools.partial(
        pl.kernel,
        mesh=mesh,
        out_type=jax.ShapeDtypeStruct((B, D), jnp.float32),
        scratch_types=[
            pltpu.VMEM((b_per_w,), jnp.int32),
            pltpu.VMEM((C, D), jnp.float32),
            pltpu.VMEM((_L,), jnp.int32),
            pltpu.SemaphoreType.DMA,
        ],
    )
    def k(x_hbm, idx_hbm, shift_hbm, out_hbm, idx_v, rows_v, shift_v, sem):
        wid = lax.axis_index("s") * _NC + lax.axis_index("c")
        base = wid * b_per_w
        pltpu.sync_copy(idx_hbm.at[pl.ds(base, b_per_w)], idx_v)
        pltpu.sync_copy(shift_hbm, shift_v)
        sh = shift_v[...]

        def adj(i, carry):
            v = idx_v[pl.ds(i * _L, _L)]
            v = v - sh
            v = jnp.where(v < 0, v + N, v)
            idx_v[pl.ds(i * _L, _L)] = v
            return carry

        lax.fori_loop(0, b_per_w // _L, adj, 0)

        def chunk(g, carry):
            pltpu.async_copy(
                x_hbm.at[idx_v.at[pl.ds(g * C, C)]], rows_v, sem
            ).wait()
            pltpu.sync_copy(rows_v, out_hbm.at[pl.ds(base + g * C, C)])
            return carry

        lax.fori_loop(0, n_chunks, chunk, 0)

    return k


def kernel(x, idx, shift):
    N, D = x.shape
    B = idx.shape[0]
    shift_vec = jnp.full(
        (_L,), jnp.asarray(shift, jnp.int32) % jnp.int32(N), dtype=jnp.int32
    )
    return _make_gather(N, D, B)(x, idx.astype(jnp.int32), shift_vec)


# speedup vs baseline: 1.8125x; 1.4461x over previous
import functools

import jax
import jax.numpy as jnp
from jax import lax
from jax.experimental import pallas as pl
from jax.experimental.pallas import tpu as pltpu
from jax.experimental.pallas import tpu_sc as plsc

_NC = 2
_NS = 16
_NW = _NC * _NS
_L = 16


@functools.lru_cache(maxsize=None)
def _make_gather(N, D, B):
    b_per_w = B // _NW
    C = 128
    n_chunks = b_per_w // C
    NBUF = 4
    n_passes = n_chunks // NBUF
    assert n_chunks % NBUF == 0 and n_passes >= 2
    mesh = plsc.VectorSubcoreMesh(core_axis_name="c", subcore_axis_name="s")

    @functools.partial(
        pl.kernel,
        mesh=mesh,
        out_type=jax.ShapeDtypeStruct((B, D), jnp.float32),
        scratch_types=[
            pltpu.VMEM((b_per_w,), jnp.int32),
            pltpu.VMEM((_L,), jnp.int32),
        ]
        + [pltpu.VMEM((C, D), jnp.float32) for _ in range(NBUF)]
        + [pltpu.SemaphoreType.DMA for _ in range(2 * NBUF)],
    )
    def k(x_hbm, idx_hbm, shift_hbm, out_hbm, idx_v, shift_v, *scratch):
        bufs = scratch[:NBUF]
        gsems = scratch[NBUF:2 * NBUF]
        ssems = scratch[2 * NBUF:]
        wid = lax.axis_index("s") * _NC + lax.axis_index("c")
        base = wid * b_per_w
        pltpu.sync_copy(idx_hbm.at[pl.ds(base, b_per_w)], idx_v)
        pltpu.sync_copy(shift_hbm, shift_v)
        sh = shift_v[...]

        def adjust(g):
            for j in range(C // _L):
                sl = pl.ds(g * C + j * _L, _L)
                v = idx_v[sl] - sh
                idx_v[sl] = jnp.where(v < 0, v + N, v)

        def start_gather(g, b):
            pltpu.async_copy(
                x_hbm.at[idx_v.at[pl.ds(g * C, C)]], bufs[b], gsems[b]
            )

        def start_scatter(g, b):
            pltpu.async_copy(
                bufs[b], out_hbm.at[pl.ds(base + g * C, C)], ssems[b]
            )

        def wait_gather(g, b):
            pltpu.make_async_copy(
                x_hbm.at[idx_v.at[pl.ds(g * C, C)]], bufs[b], gsems[b]
            ).wait()

        def wait_scatter(g, b):
            pltpu.make_async_copy(
                bufs[b], out_hbm.at[pl.ds(base + g * C, C)], ssems[b]
            ).wait()

        for b in range(NBUF):
            adjust(b)
            start_gather(b, b)

        def body(p, carry):
            for b in range(NBUF):
                g = p * NBUF + b
                wait_gather(g, b)
                start_scatter(g, b)
                adjust(g + NBUF)
                wait_scatter(g, b)
                start_gather(g + NBUF, b)
            return carry

        lax.fori_loop(0, n_passes - 1, body, 0)

        g0 = (n_passes - 1) * NBUF
        for b in range(NBUF):
            wait_gather(g0 + b, b)
            start_scatter(g0 + b, b)
        for b in range(NBUF):
            wait_scatter(g0 + b, b)

    return k


def kernel(x, idx, shift):
    N, D = x.shape
    B = idx.shape[0]
    shift_vec = jnp.full(
        (_L,), jnp.asarray(shift, jnp.int32) % jnp.int32(N), dtype=jnp.int32
    )
    return _make_gather(N, D, B)(x, idx.astype(jnp.int32), shift_vec)
